# Initial kernel scaffold; baseline (speedup 1.0000x reference)
import jax
import jax.numpy as jnp
from jax.experimental import pallas as pl


def kernel(x, edge_index, batch_index, W1, b1, W2, b2, W3, b3):
    raise NotImplementedError("write your pallas kernel here")


# speedup vs baseline: 7.7319x; 7.7319x over previous
import functools

import jax
import jax.numpy as jnp
from jax import lax
from jax.experimental import pallas as pl
from jax.experimental.pallas import tpu as pltpu
from jax.experimental.pallas import tpu_sc as plsc

N = 10000
E = 320000
F = 128
H = 64
G = 64
HW = 128
NC = 2
NS = 16
NW = NC * NS
CHUNK = 128
NJ = 80
EPAD = NW * NJ * CHUNK
NP = 10112
RPT = NP // NS
DEGW = 16

_mesh = plsc.VectorSubcoreMesh(core_axis_name="c", subcore_axis_name="s")


def _deg_body(dst_hbm, ones_hbm, zeros_hbm, parts_hbm, dst_v, ones_v, acc_sh):
    c = lax.axis_index("c")
    s = lax.axis_index("s")
    wid = s * NC + c
    pltpu.sync_copy(dst_hbm.at[wid], dst_v)
    pltpu.sync_copy(ones_hbm, ones_v)
    row0 = s * RPT
    pltpu.sync_copy(zeros_hbm, acc_sh.at[pl.ds(row0, RPT)])
    plsc.subcore_barrier()

    def body(j, carry):
        pltpu.sync_copy(ones_v, acc_sh.at[dst_v.at[j]], add=True)
        return carry

    lax.fori_loop(0, NJ, body, 0)
    plsc.subcore_barrier()
    pltpu.sync_copy(acc_sh.at[pl.ds(row0, RPT)],
                    parts_hbm.at[c, pl.ds(row0, RPT)])


_deg_call = functools.partial(
    pl.kernel,
    mesh=_mesh,
    out_type=jax.ShapeDtypeStruct((NC, NP, DEGW), jnp.float32),
    scratch_types=[
        pltpu.VMEM((NJ, CHUNK), jnp.int32),
        pltpu.VMEM((CHUNK, DEGW), jnp.float32),
        pltpu.VMEM_SHARED((NP, DEGW), jnp.float32),
    ],
)(_deg_body)


def _agg_body(u_hbm, src_hbm, dst_hbm, zeros_hbm, parts_hbm,
              src_v, dst_v, buf_v, acc_sh):
    c = lax.axis_index("c")
    s = lax.axis_index("s")
    wid = s * NC + c
    pltpu.sync_copy(src_hbm.at[wid], src_v)
    pltpu.sync_copy(dst_hbm.at[wid], dst_v)
    row0 = s * RPT
    pltpu.sync_copy(zeros_hbm, acc_sh.at[pl.ds(row0, RPT)])
    plsc.subcore_barrier()

    def body(j, carry):
        pltpu.sync_copy(u_hbm.at[src_v.at[j]], buf_v)
        pltpu.sync_copy(buf_v, acc_sh.at[dst_v.at[j]], add=True)
        return carry

    lax.fori_loop(0, NJ, body, 0)
    plsc.subcore_barrier()
    pltpu.sync_copy(acc_sh.at[pl.ds(row0, RPT)],
                    parts_hbm.at[c, pl.ds(row0, RPT)])


_agg_call = functools.partial(
    pl.kernel,
    mesh=_mesh,
    out_type=jax.ShapeDtypeStruct((NC, NP, HW), jnp.float32),
    scratch_types=[
        pltpu.VMEM((NJ, CHUNK), jnp.int32),
        pltpu.VMEM((NJ, CHUNK), jnp.int32),
        pltpu.VMEM((CHUNK, HW), jnp.float32),
        pltpu.VMEM_SHARED((NP, HW), jnp.float32),
    ],
)(_agg_body)


def _lin1_body(x_ref, w1_ref, d0_ref, d1_ref, u1_ref, dinv_ref):
    deg = 1.0 + d0_ref[:N, 0:1] + d1_ref[:N, 0:1]
    dinv = lax.rsqrt(deg)
    dinv_ref[...] = dinv
    u1_ref[...] = dinv * jnp.dot(x_ref[...], w1_ref[...],
                                 preferred_element_type=jnp.float32)


_lin1 = pl.pallas_call(
    _lin1_body,
    out_shape=(jax.ShapeDtypeStruct((N, HW), jnp.float32),
               jax.ShapeDtypeStruct((N, 1), jnp.float32)),
)


def _lin2_body(p0_ref, p1_ref, u1_ref, dinv_ref, b1_ref, w2_ref, u2_ref):
    sagg = p0_ref[:N, :H] + p1_ref[:N, :H]
    dinv = dinv_ref[...]
    h = jnp.maximum(dinv * (sagg + u1_ref[:, :H]) + b1_ref[...], 0.0)
    u2_ref[...] = dinv * jnp.dot(h, w2_ref[...],
                                 preferred_element_type=jnp.float32)


_lin2 = pl.pallas_call(
    _lin2_body,
    out_shape=jax.ShapeDtypeStruct((N, HW), jnp.float32),
)


def _head_body(p0_ref, p1_ref, u2_ref, dinv_ref, b2_ref, batch_ref,
               w3_ref, b3_ref, out_ref, h_ref, pooled_ref):
    sagg = p0_ref[:N, :H] + p1_ref[:N, :H]
    h_ref[...] = jnp.maximum(
        dinv_ref[...] * (sagg + u2_ref[:, :H]) + b2_ref[...], 0.0)
    bidx = batch_ref[...]

    def body(g, carry):
        m = jnp.where(bidx == g, h_ref[...], -jnp.inf)
        pooled_ref[pl.ds(g, 1), :] = jnp.max(m, axis=0, keepdims=True)
        return carry

    lax.fori_loop(0, G, body, 0)
    out_ref[...] = jnp.dot(pooled_ref[...], w3_ref[...],
                           preferred_element_type=jnp.float32) + b3_ref[...]


_head = pl.pallas_call(
    _head_body,
    out_shape=jax.ShapeDtypeStruct((G, 1), jnp.float32),
    scratch_shapes=[pltpu.VMEM((N, H), jnp.float32),
                    pltpu.VMEM((G, H), jnp.float32)],
)


def kernel(x, edge_index, batch_index, W1, b1, W2, b2, W3, b3):
    src = edge_index[0]
    dst = edge_index[1]
    srcp = jnp.pad(src, (0, EPAD - E)).reshape(NW, NJ, CHUNK)
    dstp = jnp.pad(dst, (0, EPAD - E),
                   constant_values=N).reshape(NW, NJ, CHUNK)
    ones = jnp.ones((CHUNK, DEGW), jnp.float32)
    zeros_deg = jnp.zeros((RPT, DEGW), jnp.float32)
    zeros_h = jnp.zeros((RPT, HW), jnp.float32)
    W1p = jnp.pad(W1, ((0, 0), (0, HW - H)))
    W2p = jnp.pad(W2, ((0, 0), (0, HW - H)))

    deg_parts = _deg_call(dstp, ones, zeros_deg)
    u1, dinv = _lin1(x, W1p, deg_parts[0], deg_parts[1])
    s1 = _agg_call(u1, srcp, dstp, zeros_h)
    u2 = _lin2(s1[0], s1[1], u1, dinv, b1.reshape(1, H), W2p)
    s2 = _agg_call(u2, srcp, dstp, zeros_h)
    out = _head(s2[0], s2[1], u2, dinv, b2.reshape(1, H),
                batch_index.reshape(N, 1), W3, b3.reshape(1, 1))
    return out
